# baseline (speedup 1.0000x reference)
import math

import jax
import jax.numpy as jnp
import numpy as np
from jax.experimental import pallas as pl
from jax.experimental.pallas import tpu as pltpu

_B, _J, _H, _W = 256, 16, 64, 64
_N = 16
_K = _H * _W
_R = 64
_C = 512
_R2 = 128
_NB = _R2 // _J
_EMB = 64
_TINY = np.float32(np.finfo(np.float32).tiny)

_KS = (np.uint32(0), np.uint32(42), np.uint32(0x1BD11BDA ^ 42))
_ROTS = (13, 15, 26, 6, 17, 29, 16, 24)


def _threefry_xor(lo42):
    x1 = lo42
    x0 = x1
    x1 = ((x1 << np.uint32(13)) | (x1 >> np.uint32(19))) ^ x0
    first = True
    for i in range(5):
        for j in range(4):
            if first:
                first = False
                continue
            r = _ROTS[(i % 2) * 4 + j]
            x0 = x0 + x1
            x1 = (x1 << np.uint32(r)) | (x1 >> np.uint32(32 - r))
            x1 = x1 ^ x0
        ka = _KS[(i + 1) % 3]
        kb = np.uint32(int(_KS[(i + 2) % 3]) + i + 1)
        if int(ka) != 0:
            x0 = x0 + ka
        x1 = x1 + kb
    return x0 ^ x1


def _sample_body(hm_ref, idx_ref, pr_ref):
    pid = pl.program_id(0)

    p = jnp.maximum(hm_ref[...], 0.0)
    logits = jnp.where(p > 0.0, jnp.log(jnp.where(p > 0.0, p, 1.0)),
                       -jnp.inf)

    row_u = jax.lax.broadcasted_iota(jnp.uint32, (_R, _C), 0)
    lane_u = jax.lax.broadcasted_iota(jnp.uint32, (_R, _C), 1)
    lane_i = jax.lax.broadcasted_iota(jnp.int32, (_R, _C), 1)
    base42 = (row_u + pid.astype(jnp.uint32) * np.uint32(_R)) * np.uint32(_K) \
        + lane_u + np.uint32(42)
    for s in range(_N):
        lo0 = base42 + np.uint32(s << 24)

        def chunk_v(c):
            bits = _threefry_xor(lo0 + np.uint32(c * _C))
            f = pltpu.bitcast((bits >> np.uint32(9)) | np.uint32(0x3F800000),
                              jnp.float32) - 1.0
            u = jnp.maximum(_TINY, f)
            return logits[:, c * _C:(c + 1) * _C] - jnp.log(-jnp.log(u))

        m_run = chunk_v(0)
        c_run = jnp.zeros((_R, _C), jnp.int32)
        p_run = p[:, 0:_C]
        for c in range(1, _K // _C):
            v = chunk_v(c)
            better = v > m_run
            m_run = jnp.where(better, v, m_run)
            c_run = jnp.where(better, c, c_run)
            p_run = jnp.where(better, p[:, c * _C:(c + 1) * _C], p_run)
        i_run = c_run * _C + lane_i
        m = jnp.max(m_run, axis=1, keepdims=True)
        idx = jnp.min(jnp.where(m_run == m, i_run, _K), axis=1, keepdims=True)
        pr = jnp.sum(jnp.where(i_run == idx, p_run, 0.0), axis=1,
                     keepdims=True)
        idx_ref[:, s:s + 1] = idx.astype(jnp.float32)
        pr_ref[:, s:s + 1] = pr


def _emb_body(idx_ref, pr_ref, jt_ref, c_ref, out_ref):
    idxs = idx_ref[...]
    prs = pr_ref[...]

    q = jnp.floor(idxs * np.float32(1.0 / _W))
    x = idxs - np.float32(_W) * q
    y = q

    bi = jax.lax.broadcasted_iota(jnp.int32, (_NB, _R2), 0)
    ri = jax.lax.broadcasted_iota(jnp.int32, (_NB, _R2), 1)
    bsel = (ri // _J == bi).astype(jnp.float32)
    bselT = bsel.T

    def bsum(a):
        return jnp.sum(jnp.dot(bsel, a, preferred_element_type=jnp.float32),
                       axis=1, keepdims=True)

    inv_jn = np.float32(1.0 / (_J * _N))
    mean_x = bsum(x) * inv_jn
    mean_y = bsum(y) * inv_jn
    xc = x - jnp.dot(bselT, mean_x, preferred_element_type=jnp.float32)
    yc = y - jnp.dot(bselT, mean_y, preferred_element_type=jnp.float32)

    n_tot = np.float32(2 * _J * _N)
    s1 = bsum(xc + yc)
    s2 = bsum(xc * xc + yc * yc)
    m2 = s1 / n_tot
    var = (s2 - n_tot * m2 * m2) * np.float32(1.0 / (2 * _J * _N - 1))
    inv_std = jax.lax.rsqrt(var)
    inv_rows = jnp.dot(bselT, inv_std, preferred_element_type=jnp.float32)
    xn = xc * inv_rows
    yn = yc * inv_rows

    fx = c_ref[0:1, :]
    fy = c_ref[1:2, :]
    smask = c_ref[2:3, :] > 0.5
    jtb = jnp.concatenate([jt_ref[...]] * _NB, axis=0)

    for s in range(_N):
        arg = xn[:, s:s + 1] * fx + yn[:, s:s + 1] * fy
        val = jnp.where(smask, jnp.sin(arg), jnp.cos(arg))
        piece = val * prs[:, s:s + 1] + jtb
        out_ref[:, s, :, :] = piece.reshape(_NB, _J, 2 * _EMB)


def _consts():
    half = _EMB // 2
    scale = math.log(10000.0) / (half - 1)
    freqs = np.exp(np.arange(half, dtype=np.float32) * -scale)
    c = np.zeros((8, 128), np.float32)
    c[0, 0:32] = freqs
    c[0, 32:64] = freqs
    c[1, 64:96] = freqs
    c[1, 96:128] = freqs
    c[2, 0:32] = 1.0
    c[2, 64:96] = 1.0
    return jnp.asarray(c)


def kernel(heatmap, joint_table):
    flat = heatmap.reshape(_B * _J, _K)
    idxs, prs = pl.pallas_call(
        _sample_body,
        grid=(_B * _J // _R,),
        in_specs=[pl.BlockSpec((_R, _K), lambda i: (i, 0))],
        out_specs=[pl.BlockSpec((_R, _N), lambda i: (i, 0)),
                   pl.BlockSpec((_R, _N), lambda i: (i, 0))],
        out_shape=[jax.ShapeDtypeStruct((_B * _J, _N), jnp.float32),
                   jax.ShapeDtypeStruct((_B * _J, _N), jnp.float32)],
        compiler_params=pltpu.CompilerParams(
            dimension_semantics=("parallel",)),
    )(flat)
    return pl.pallas_call(
        _emb_body,
        grid=(_B * _J // _R2,),
        in_specs=[
            pl.BlockSpec((_R2, _N), lambda i: (i, 0)),
            pl.BlockSpec((_R2, _N), lambda i: (i, 0)),
            pl.BlockSpec((_J, 2 * _EMB), lambda i: (0, 0)),
            pl.BlockSpec((8, 128), lambda i: (0, 0)),
        ],
        out_specs=pl.BlockSpec((_NB, _N, _J, 2 * _EMB),
                               lambda i: (i, 0, 0, 0)),
        out_shape=jax.ShapeDtypeStruct((_B, _N, _J, 2 * _EMB), jnp.float32),
        compiler_params=pltpu.CompilerParams(
            dimension_semantics=("parallel",)),
    )(idxs, prs, joint_table, _consts())


# speedup vs baseline: 1.0544x; 1.0544x over previous
import math

import jax
import jax.numpy as jnp
import numpy as np
from jax.experimental import pallas as pl
from jax.experimental.pallas import tpu as pltpu

_B, _J, _H, _W = 256, 16, 64, 64
_N = 16
_K = _H * _W
_R = 128
_C = 512
_R2 = 128
_NB = _R2 // _J
_EMB = 64
_TINY = np.float32(np.finfo(np.float32).tiny)

_KS = (np.uint32(0), np.uint32(42), np.uint32(0x1BD11BDA ^ 42))
_ROTS = (13, 15, 26, 6, 17, 29, 16, 24)


def _threefry_xor(lo42):
    x1 = lo42
    x0 = x1
    x1 = ((x1 << np.uint32(13)) | (x1 >> np.uint32(19))) ^ x0
    first = True
    for i in range(5):
        for j in range(4):
            if first:
                first = False
                continue
            r = _ROTS[(i % 2) * 4 + j]
            x0 = x0 + x1
            x1 = (x1 << np.uint32(r)) | (x1 >> np.uint32(32 - r))
            x1 = x1 ^ x0
        ka = _KS[(i + 1) % 3]
        kb = np.uint32(int(_KS[(i + 2) % 3]) + i + 1)
        if int(ka) != 0:
            x0 = x0 + ka
        x1 = x1 + kb
    return x0 ^ x1


def _sample_body(hm_ref, idx_ref, pr_ref):
    pid = pl.program_id(0)

    p = jnp.maximum(hm_ref[...], 0.0)
    logits = jnp.where(p > 0.0, jnp.log(jnp.where(p > 0.0, p, 1.0)),
                       -jnp.inf)

    row_u = jax.lax.broadcasted_iota(jnp.uint32, (_R, _C), 0)
    lane_u = jax.lax.broadcasted_iota(jnp.uint32, (_R, _C), 1)
    lane_i = jax.lax.broadcasted_iota(jnp.int32, (_R, _C), 1)
    base42 = (row_u + pid.astype(jnp.uint32) * np.uint32(_R)) * np.uint32(_K) \
        + lane_u + np.uint32(42)
    for s in range(_N):
        lo0 = base42 + np.uint32(s << 24)

        def chunk_v(c):
            bits = _threefry_xor(lo0 + np.uint32(c * _C))
            f = pltpu.bitcast((bits >> np.uint32(9)) | np.uint32(0x3F800000),
                              jnp.float32) - 1.0
            u = jnp.maximum(_TINY, f)
            return logits[:, c * _C:(c + 1) * _C] - jnp.log(-jnp.log(u))

        m_run = chunk_v(0)
        c_run = jnp.zeros((_R, _C), jnp.int32)
        p_run = p[:, 0:_C]
        for c in range(1, _K // _C):
            v = chunk_v(c)
            better = v > m_run
            m_run = jnp.where(better, v, m_run)
            c_run = jnp.where(better, c, c_run)
            p_run = jnp.where(better, p[:, c * _C:(c + 1) * _C], p_run)
        i_run = c_run * _C + lane_i
        m = jnp.max(m_run, axis=1, keepdims=True)
        idx = jnp.min(jnp.where(m_run == m, i_run, _K), axis=1, keepdims=True)
        pr = jnp.sum(jnp.where(i_run == idx, p_run, 0.0), axis=1,
                     keepdims=True)
        idx_ref[:, s:s + 1] = idx.astype(jnp.float32)
        pr_ref[:, s:s + 1] = pr


def _emb_body(idx_ref, pr_ref, jt_ref, c_ref, out_ref):
    idxs = idx_ref[...]
    prs = pr_ref[...]

    q = jnp.floor(idxs * np.float32(1.0 / _W))
    x = idxs - np.float32(_W) * q
    y = q

    bi = jax.lax.broadcasted_iota(jnp.int32, (_NB, _R2), 0)
    ri = jax.lax.broadcasted_iota(jnp.int32, (_NB, _R2), 1)
    bsel = (ri // _J == bi).astype(jnp.float32)
    bselT = bsel.T

    def bsum(a):
        return jnp.sum(jnp.dot(bsel, a, preferred_element_type=jnp.float32),
                       axis=1, keepdims=True)

    inv_jn = np.float32(1.0 / (_J * _N))
    mean_x = bsum(x) * inv_jn
    mean_y = bsum(y) * inv_jn
    xc = x - jnp.dot(bselT, mean_x, preferred_element_type=jnp.float32)
    yc = y - jnp.dot(bselT, mean_y, preferred_element_type=jnp.float32)

    n_tot = np.float32(2 * _J * _N)
    s1 = bsum(xc + yc)
    s2 = bsum(xc * xc + yc * yc)
    m2 = s1 / n_tot
    var = (s2 - n_tot * m2 * m2) * np.float32(1.0 / (2 * _J * _N - 1))
    inv_std = jax.lax.rsqrt(var)
    inv_rows = jnp.dot(bselT, inv_std, preferred_element_type=jnp.float32)
    xn = xc * inv_rows
    yn = yc * inv_rows

    fx = c_ref[0:1, :]
    fy = c_ref[1:2, :]
    smask = c_ref[2:3, :] > 0.5
    jtb = jnp.concatenate([jt_ref[...]] * _NB, axis=0)

    for s in range(_N):
        arg = xn[:, s:s + 1] * fx + yn[:, s:s + 1] * fy
        val = jnp.where(smask, jnp.sin(arg), jnp.cos(arg))
        piece = val * prs[:, s:s + 1] + jtb
        out_ref[:, s, :, :] = piece.reshape(_NB, _J, 2 * _EMB)


def _consts():
    half = _EMB // 2
    scale = math.log(10000.0) / (half - 1)
    freqs = np.exp(np.arange(half, dtype=np.float32) * -scale)
    c = np.zeros((8, 128), np.float32)
    c[0, 0:32] = freqs
    c[0, 32:64] = freqs
    c[1, 64:96] = freqs
    c[1, 96:128] = freqs
    c[2, 0:32] = 1.0
    c[2, 64:96] = 1.0
    return jnp.asarray(c)


def kernel(heatmap, joint_table):
    flat = heatmap.reshape(_B * _J, _K)
    idxs, prs = pl.pallas_call(
        _sample_body,
        grid=(_B * _J // _R,),
        in_specs=[pl.BlockSpec((_R, _K), lambda i: (i, 0))],
        out_specs=[pl.BlockSpec((_R, _N), lambda i: (i, 0)),
                   pl.BlockSpec((_R, _N), lambda i: (i, 0))],
        out_shape=[jax.ShapeDtypeStruct((_B * _J, _N), jnp.float32),
                   jax.ShapeDtypeStruct((_B * _J, _N), jnp.float32)],
        compiler_params=pltpu.CompilerParams(
            dimension_semantics=("parallel",)),
    )(flat)
    return pl.pallas_call(
        _emb_body,
        grid=(_B * _J // _R2,),
        in_specs=[
            pl.BlockSpec((_R2, _N), lambda i: (i, 0)),
            pl.BlockSpec((_R2, _N), lambda i: (i, 0)),
            pl.BlockSpec((_J, 2 * _EMB), lambda i: (0, 0)),
            pl.BlockSpec((8, 128), lambda i: (0, 0)),
        ],
        out_specs=pl.BlockSpec((_NB, _N, _J, 2 * _EMB),
                               lambda i: (i, 0, 0, 0)),
        out_shape=jax.ShapeDtypeStruct((_B, _N, _J, 2 * _EMB), jnp.float32),
        compiler_params=pltpu.CompilerParams(
            dimension_semantics=("parallel",)),
    )(idxs, prs, joint_table, _consts())


# speedup vs baseline: 1.0812x; 1.0254x over previous
import math

import jax
import jax.numpy as jnp
import numpy as np
from jax.experimental import pallas as pl
from jax.experimental.pallas import tpu as pltpu

_B, _J, _H, _W = 256, 16, 64, 64
_N = 16
_K = _H * _W
_R = 256
_C = 512
_R2 = 128
_NB = _R2 // _J
_EMB = 64
_TINY = np.float32(np.finfo(np.float32).tiny)

_KS = (np.uint32(0), np.uint32(42), np.uint32(0x1BD11BDA ^ 42))
_ROTS = (13, 15, 26, 6, 17, 29, 16, 24)


def _threefry_xor(lo42):
    x1 = lo42
    x0 = x1
    x1 = ((x1 << np.uint32(13)) | (x1 >> np.uint32(19))) ^ x0
    first = True
    for i in range(5):
        for j in range(4):
            if first:
                first = False
                continue
            r = _ROTS[(i % 2) * 4 + j]
            x0 = x0 + x1
            x1 = (x1 << np.uint32(r)) | (x1 >> np.uint32(32 - r))
            x1 = x1 ^ x0
        ka = _KS[(i + 1) % 3]
        kb = np.uint32(int(_KS[(i + 2) % 3]) + i + 1)
        if int(ka) != 0:
            x0 = x0 + ka
        x1 = x1 + kb
    return x0 ^ x1


def _sample_body(hm_ref, idx_ref, pr_ref):
    pid = pl.program_id(0)

    p = jnp.maximum(hm_ref[...], 0.0)
    logits = jnp.where(p > 0.0, jnp.log(jnp.where(p > 0.0, p, 1.0)),
                       -jnp.inf)

    row_u = jax.lax.broadcasted_iota(jnp.uint32, (_R, _C), 0)
    lane_u = jax.lax.broadcasted_iota(jnp.uint32, (_R, _C), 1)
    lane_i = jax.lax.broadcasted_iota(jnp.int32, (_R, _C), 1)
    base42 = (row_u + pid.astype(jnp.uint32) * np.uint32(_R)) * np.uint32(_K) \
        + lane_u + np.uint32(42)
    for s in range(_N):
        lo0 = base42 + np.uint32(s << 24)

        def chunk_v(c):
            bits = _threefry_xor(lo0 + np.uint32(c * _C))
            f = pltpu.bitcast((bits >> np.uint32(9)) | np.uint32(0x3F800000),
                              jnp.float32) - 1.0
            u = jnp.maximum(_TINY, f)
            return logits[:, c * _C:(c + 1) * _C] - jnp.log(-jnp.log(u))

        m_run = chunk_v(0)
        c_run = jnp.zeros((_R, _C), jnp.int32)
        p_run = p[:, 0:_C]
        for c in range(1, _K // _C):
            v = chunk_v(c)
            better = v > m_run
            m_run = jnp.where(better, v, m_run)
            c_run = jnp.where(better, c, c_run)
            p_run = jnp.where(better, p[:, c * _C:(c + 1) * _C], p_run)
        i_run = c_run * _C + lane_i
        m = jnp.max(m_run, axis=1, keepdims=True)
        idx = jnp.min(jnp.where(m_run == m, i_run, _K), axis=1, keepdims=True)
        pr = jnp.sum(jnp.where(i_run == idx, p_run, 0.0), axis=1,
                     keepdims=True)
        idx_ref[:, s:s + 1] = idx.astype(jnp.float32)
        pr_ref[:, s:s + 1] = pr


def _emb_body(idx_ref, pr_ref, jt_ref, c_ref, out_ref):
    idxs = idx_ref[...]
    prs = pr_ref[...]

    q = jnp.floor(idxs * np.float32(1.0 / _W))
    x = idxs - np.float32(_W) * q
    y = q

    bi = jax.lax.broadcasted_iota(jnp.int32, (_NB, _R2), 0)
    ri = jax.lax.broadcasted_iota(jnp.int32, (_NB, _R2), 1)
    bsel = (ri // _J == bi).astype(jnp.float32)
    bselT = bsel.T

    def bsum(a):
        return jnp.sum(jnp.dot(bsel, a, preferred_element_type=jnp.float32),
                       axis=1, keepdims=True)

    inv_jn = np.float32(1.0 / (_J * _N))
    mean_x = bsum(x) * inv_jn
    mean_y = bsum(y) * inv_jn
    xc = x - jnp.dot(bselT, mean_x, preferred_element_type=jnp.float32)
    yc = y - jnp.dot(bselT, mean_y, preferred_element_type=jnp.float32)

    n_tot = np.float32(2 * _J * _N)
    s1 = bsum(xc + yc)
    s2 = bsum(xc * xc + yc * yc)
    m2 = s1 / n_tot
    var = (s2 - n_tot * m2 * m2) * np.float32(1.0 / (2 * _J * _N - 1))
    inv_std = jax.lax.rsqrt(var)
    inv_rows = jnp.dot(bselT, inv_std, preferred_element_type=jnp.float32)
    xn = xc * inv_rows
    yn = yc * inv_rows

    fx = c_ref[0:1, :]
    fy = c_ref[1:2, :]
    smask = c_ref[2:3, :] > 0.5
    jtb = jnp.concatenate([jt_ref[...]] * _NB, axis=0)

    for s in range(_N):
        arg = xn[:, s:s + 1] * fx + yn[:, s:s + 1] * fy
        val = jnp.where(smask, jnp.sin(arg), jnp.cos(arg))
        piece = val * prs[:, s:s + 1] + jtb
        out_ref[:, s, :, :] = piece.reshape(_NB, _J, 2 * _EMB)


def _consts():
    half = _EMB // 2
    scale = math.log(10000.0) / (half - 1)
    freqs = np.exp(np.arange(half, dtype=np.float32) * -scale)
    c = np.zeros((8, 128), np.float32)
    c[0, 0:32] = freqs
    c[0, 32:64] = freqs
    c[1, 64:96] = freqs
    c[1, 96:128] = freqs
    c[2, 0:32] = 1.0
    c[2, 64:96] = 1.0
    return jnp.asarray(c)


def kernel(heatmap, joint_table):
    flat = heatmap.reshape(_B * _J, _K)
    idxs, prs = pl.pallas_call(
        _sample_body,
        grid=(_B * _J // _R,),
        in_specs=[pl.BlockSpec((_R, _K), lambda i: (i, 0))],
        out_specs=[pl.BlockSpec((_R, _N), lambda i: (i, 0)),
                   pl.BlockSpec((_R, _N), lambda i: (i, 0))],
        out_shape=[jax.ShapeDtypeStruct((_B * _J, _N), jnp.float32),
                   jax.ShapeDtypeStruct((_B * _J, _N), jnp.float32)],
        compiler_params=pltpu.CompilerParams(
            dimension_semantics=("parallel",)),
    )(flat)
    return pl.pallas_call(
        _emb_body,
        grid=(_B * _J // _R2,),
        in_specs=[
            pl.BlockSpec((_R2, _N), lambda i: (i, 0)),
            pl.BlockSpec((_R2, _N), lambda i: (i, 0)),
            pl.BlockSpec((_J, 2 * _EMB), lambda i: (0, 0)),
            pl.BlockSpec((8, 128), lambda i: (0, 0)),
        ],
        out_specs=pl.BlockSpec((_NB, _N, _J, 2 * _EMB),
                               lambda i: (i, 0, 0, 0)),
        out_shape=jax.ShapeDtypeStruct((_B, _N, _J, 2 * _EMB), jnp.float32),
        compiler_params=pltpu.CompilerParams(
            dimension_semantics=("parallel",)),
    )(idxs, prs, joint_table, _consts())
